# Initial kernel scaffold; baseline (speedup 1.0000x reference)
import jax
import jax.numpy as jnp
from jax.experimental import pallas as pl


def kernel(x, fc_weight, bias):
    raise NotImplementedError("write your pallas kernel here")


# speedup vs baseline: 1.1010x; 1.1010x over previous
import jax
import jax.numpy as jnp
from jax import lax
from jax.experimental import pallas as pl
from jax.experimental.pallas import tpu as pltpu
from jax.experimental.pallas import tpu_sc as plsc

_BATCH = 4096
_FIELDS = 26
_NC = 2
_NS = 16
_NW = _NC * _NS
_BPW = _BATCH // _NW
_L = 16


def _sc_body(xt_hbm, w_hbm, out_hbm, idx_v, vals_v, out_v, sem):
    wid = lax.axis_index("s") * _NC + lax.axis_index("c")
    pltpu.sync_copy(xt_hbm.at[wid], idx_v)
    copies = [
        pltpu.async_copy(w_hbm.at[idx_v.at[j]], vals_v.at[j], sem)
        for j in range(_FIELDS)
    ]
    for c in copies:
        c.wait()
    for chunk in range(_BPW // _L):
        sl = pl.ds(chunk * _L, _L)
        acc = vals_v[0, sl]
        for j in range(1, _FIELDS):
            acc = acc + vals_v[j, sl]
        out_v[sl] = acc
    pltpu.sync_copy(out_v, out_hbm.at[pl.ds(wid * _BPW, _BPW)])


def kernel(x, fc_weight, bias):
    xt = jnp.transpose(
        x.astype(jnp.int32).reshape(_NW, _BPW, _FIELDS), (0, 2, 1)
    )
    w = fc_weight.reshape(-1)
    mesh = plsc.VectorSubcoreMesh(core_axis_name="c", subcore_axis_name="s")
    out = pl.kernel(
        _sc_body,
        out_type=jax.ShapeDtypeStruct((_BATCH,), jnp.float32),
        mesh=mesh,
        scratch_types=[
            pltpu.VMEM((_FIELDS, _BPW), jnp.int32),
            pltpu.VMEM((_FIELDS, _BPW), jnp.float32),
            pltpu.VMEM((_BPW,), jnp.float32),
            pltpu.SemaphoreType.DMA,
        ],
    )(xt, w)
    return out.reshape(_BATCH, 1) + bias
